# baseline (speedup 1.0000x reference)
import functools

import jax
import jax.numpy as jnp
from jax import lax
from jax.experimental import pallas as pl
from jax.experimental.pallas import tpu as pltpu, tpu_sc as plsc

DIM = 2048
N_EXPERTS = 64
TOPK = 8
N_GROUPS = 8
EPG = N_EXPERTS // N_GROUPS
TOPK_GROUPS = 4
ROUTE_SCALE = 2.5
B_TOKENS = 16384
TBLK = 2048

_info = plsc.get_sparse_core_info()
_NC, _NS, _L = _info.num_cores, _info.num_subcores, _info.num_lanes
_NW = _NC * _NS
_RPW = B_TOKENS // _NW
_NT = _RPW // _L


def _scores_kernel(x_ref, w_ref, b_ref, sbt_ref):
    lt = jax.lax.dot_general(
        w_ref[...], x_ref[...], (((1,), (1,)), ((), ())),
        preferred_element_type=jnp.float32)
    sbt_ref[...] = jax.nn.sigmoid(lt) + b_ref[...]


def _scores_t(x, W, b):
    bc = b.reshape(N_EXPERTS, 1)
    return pl.pallas_call(
        _scores_kernel,
        grid=(B_TOKENS // TBLK,),
        in_specs=[
            pl.BlockSpec((TBLK, DIM), lambda i: (i, 0)),
            pl.BlockSpec((N_EXPERTS, DIM), lambda i: (0, 0)),
            pl.BlockSpec((N_EXPERTS, 1), lambda i: (0, 0)),
        ],
        out_specs=pl.BlockSpec((N_EXPERTS, TBLK), lambda i: (0, i)),
        out_shape=jax.ShapeDtypeStruct((N_EXPERTS, B_TOKENS), jnp.float32),
    )(x, W, bc)


_mesh = plsc.VectorSubcoreMesh(core_axis_name="c", subcore_axis_name="s")


@functools.partial(
    pl.kernel,
    out_type=[
        jax.ShapeDtypeStruct((B_TOKENS, TOPK), jnp.float32),
        jax.ShapeDtypeStruct((B_TOKENS, TOPK), jnp.int32),
    ],
    mesh=_mesh,
    scratch_types=[
        pltpu.VMEM((N_EXPERTS, _RPW), jnp.float32),
        pltpu.VMEM((N_EXPERTS,), jnp.float32),
        pltpu.VMEM((_RPW, TOPK), jnp.float32),
        pltpu.VMEM((_RPW, TOPK), jnp.int32),
        pltpu.SemaphoreType.DMA,
        pltpu.SemaphoreType.DMA,
    ],
    compiler_params=pltpu.CompilerParams(
        use_tc_tiling_on_sc=False, needs_layout_passes=False),
)
def _sc_route(sbt_hbm, b_hbm, wout_hbm, iout_hbm, sbloc, bloc, wloc, iloc,
              sem1, sem2):
    wid = lax.axis_index("s") * _NC + lax.axis_index("c")
    base = wid * _RPW
    cp1 = pltpu.async_copy(sbt_hbm.at[:, pl.ds(base, _RPW)], sbloc, sem1)
    cp2 = pltpu.async_copy(b_hbm, bloc, sem2)
    cp1.wait()
    cp2.wait()

    lanes = lax.iota(jnp.int32, _L)
    one = jnp.float32(1.0)
    zero = jnp.float32(0.0)

    def tile_body(t, carry):
        off = t * _L
        sb = [sbloc[e, pl.ds(off, _L)] for e in range(N_EXPERTS)]

        gs = []
        for g in range(N_GROUPS):
            v = sb[g * EPG:(g + 1) * EPG]
            hi = jnp.maximum(v[0], v[1])
            lo = jnp.minimum(v[0], v[1])
            for c in v[2:]:
                lo = jnp.maximum(lo, jnp.minimum(hi, c))
                hi = jnp.maximum(hi, c)
            gs.append(hi + lo)

        cnt = [jnp.zeros((_L,), jnp.float32) for _ in range(N_GROUPS)]
        for h in range(N_GROUPS):
            for g in range(h + 1, N_GROUPS):
                c = gs[h] >= gs[g]
                cf = jnp.where(c, one, zero)
                cnt[g] = cnt[g] + cf
                cnt[h] = cnt[h] + (one - cf)
        keep = [jnp.where(cnt[g] < TOPK_GROUPS, one, zero)
                for g in range(N_GROUPS)]

        masked = [sb[e] * keep[e // EPG] for e in range(N_EXPERTS)]

        m = [jnp.full((_L,), -1e30, jnp.float32) for _ in range(TOPK)]
        mi = [jnp.zeros((_L,), jnp.int32) for _ in range(TOPK)]
        for e in range(N_EXPERTS):
            v = masked[e]
            vi = jnp.full((_L,), e, jnp.int32)
            for k in range(TOPK):
                c = v > m[k]
                nm = jnp.where(c, v, m[k])
                v = jnp.where(c, m[k], v)
                ni = jnp.where(c, vi, mi[k])
                vi = jnp.where(c, mi[k], vi)
                m[k] = nm
                mi[k] = ni

        rows = off + lanes
        w = [plsc.load_gather(sbloc, [mi[k], rows])
             - plsc.load_gather(bloc, [mi[k]]) for k in range(TOPK)]
        wsum = w[0]
        for c in w[1:]:
            wsum = wsum + c
        scale = ROUTE_SCALE / wsum
        for k in range(TOPK):
            kcol = jnp.full((_L,), k, jnp.int32)
            plsc.store_scatter(wloc, [rows, kcol], w[k] * scale)
            plsc.store_scatter(iloc, [rows, kcol], mi[k])
        return carry

    lax.fori_loop(0, _NT, tile_body, 0)
    pltpu.sync_copy(wloc, wout_hbm.at[pl.ds(base, _RPW), :])
    pltpu.sync_copy(iloc, iout_hbm.at[pl.ds(base, _RPW), :])


def kernel(x, W, b):
    sbt = _scores_t(x, W, b)
    wts, idxs = _sc_route(sbt, b)
    return wts.astype(x.dtype), idxs


# speedup vs baseline: 1.0108x; 1.0108x over previous
import functools

import jax
import jax.numpy as jnp
from jax import lax
from jax.experimental import pallas as pl
from jax.experimental.pallas import tpu as pltpu, tpu_sc as plsc

DIM = 2048
N_EXPERTS = 64
TOPK = 8
N_GROUPS = 8
EPG = N_EXPERTS // N_GROUPS
TOPK_GROUPS = 4
ROUTE_SCALE = 2.5
B_TOKENS = 16384
TBLK = 1024

_info = plsc.get_sparse_core_info()
_NC, _NS, _L = _info.num_cores, _info.num_subcores, _info.num_lanes
_NW = _NC * _NS
_RPW = B_TOKENS // _NW
_NT = _RPW // _L


def _scores_kernel(x_ref, w_ref, b_ref, sbt_ref):
    lt = jax.lax.dot_general(
        w_ref[...], x_ref[...], (((1,), (1,)), ((), ())),
        preferred_element_type=jnp.float32)
    sbt_ref[...] = jax.nn.sigmoid(lt) + b_ref[...]


def _scores_t(x, W, b):
    bc = b.reshape(N_EXPERTS, 1)
    return pl.pallas_call(
        _scores_kernel,
        grid=(B_TOKENS // TBLK,),
        in_specs=[
            pl.BlockSpec((TBLK, DIM), lambda i: (i, 0)),
            pl.BlockSpec((N_EXPERTS, DIM), lambda i: (0, 0)),
            pl.BlockSpec((N_EXPERTS, 1), lambda i: (0, 0)),
        ],
        out_specs=pl.BlockSpec((N_EXPERTS, TBLK), lambda i: (0, i)),
        out_shape=jax.ShapeDtypeStruct((N_EXPERTS, B_TOKENS), jnp.float32),
    )(x, W, bc)


_mesh = plsc.VectorSubcoreMesh(core_axis_name="c", subcore_axis_name="s")


@functools.partial(
    pl.kernel,
    out_type=[
        jax.ShapeDtypeStruct((B_TOKENS, TOPK), jnp.float32),
        jax.ShapeDtypeStruct((B_TOKENS, TOPK), jnp.int32),
    ],
    mesh=_mesh,
    scratch_types=[
        pltpu.VMEM((N_EXPERTS, _RPW), jnp.float32),
        pltpu.VMEM((N_EXPERTS,), jnp.float32),
        pltpu.VMEM((_RPW, TOPK), jnp.float32),
        pltpu.VMEM((_RPW, TOPK), jnp.int32),
        pltpu.SemaphoreType.DMA,
        pltpu.SemaphoreType.DMA,
    ],
    compiler_params=pltpu.CompilerParams(
        use_tc_tiling_on_sc=False, needs_layout_passes=False),
)
def _sc_route(sbt_hbm, b_hbm, wout_hbm, iout_hbm, sbloc, bloc, wloc, iloc,
              sem1, sem2):
    wid = lax.axis_index("s") * _NC + lax.axis_index("c")
    base = wid * _RPW
    cp1 = pltpu.async_copy(sbt_hbm.at[:, pl.ds(base, _RPW)], sbloc, sem1)
    cp2 = pltpu.async_copy(b_hbm, bloc, sem2)
    cp1.wait()
    cp2.wait()

    lanes = lax.iota(jnp.int32, _L)
    one = jnp.float32(1.0)
    zero = jnp.float32(0.0)

    def tile_body(t, carry):
        off = t * _L
        sb = [sbloc[e, pl.ds(off, _L)] for e in range(N_EXPERTS)]

        gs = []
        for g in range(N_GROUPS):
            v = sb[g * EPG:(g + 1) * EPG]
            hi = jnp.maximum(v[0], v[1])
            lo = jnp.minimum(v[0], v[1])
            for c in v[2:]:
                lo = jnp.maximum(lo, jnp.minimum(hi, c))
                hi = jnp.maximum(hi, c)
            gs.append(hi + lo)

        cnt = [jnp.zeros((_L,), jnp.float32) for _ in range(N_GROUPS)]
        for h in range(N_GROUPS):
            for g in range(h + 1, N_GROUPS):
                c = gs[h] >= gs[g]
                cf = jnp.where(c, one, zero)
                cnt[g] = cnt[g] + cf
                cnt[h] = cnt[h] + (one - cf)
        keep = [jnp.where(cnt[g] < TOPK_GROUPS, one, zero)
                for g in range(N_GROUPS)]

        masked = [sb[e] * keep[e // EPG] for e in range(N_EXPERTS)]

        m = [jnp.full((_L,), -1e30, jnp.float32) for _ in range(TOPK)]
        mi = [jnp.zeros((_L,), jnp.int32) for _ in range(TOPK)]
        for e in range(N_EXPERTS):
            v = masked[e]
            vi = jnp.full((_L,), e, jnp.int32)
            for k in range(TOPK):
                c = v > m[k]
                nm = jnp.where(c, v, m[k])
                v = jnp.where(c, m[k], v)
                ni = jnp.where(c, vi, mi[k])
                vi = jnp.where(c, mi[k], vi)
                m[k] = nm
                mi[k] = ni

        rows = off + lanes
        w = [plsc.load_gather(sbloc, [mi[k], rows])
             - plsc.load_gather(bloc, [mi[k]]) for k in range(TOPK)]
        wsum = w[0]
        for c in w[1:]:
            wsum = wsum + c
        scale = ROUTE_SCALE / wsum
        for k in range(TOPK):
            kcol = jnp.full((_L,), k, jnp.int32)
            plsc.store_scatter(wloc, [rows, kcol], w[k] * scale)
            plsc.store_scatter(iloc, [rows, kcol], mi[k])
        return carry

    lax.fori_loop(0, _NT, tile_body, 0)
    pltpu.sync_copy(wloc, wout_hbm.at[pl.ds(base, _RPW), :])
    pltpu.sync_copy(iloc, iout_hbm.at[pl.ds(base, _RPW), :])


def kernel(x, W, b):
    sbt = _scores_t(x, W, b)
    wts, idxs = _sc_route(sbt, b)
    return wts.astype(x.dtype), idxs
